# baseline (speedup 1.0000x reference)
import jax
import jax.numpy as jnp
from jax import lax
from jax.experimental import pallas as pl
from jax.experimental.pallas import tpu as pltpu
from jax.experimental.pallas import tpu_sc as plsc

D = 2048
R = 8192
NC = 2
NS = 16
NW = NC * NS
R_PER_W = R // NW
CH2 = 8
NBUF = 3
N_CHUNKS = R_PER_W // CH2
N_ROUNDS = (N_CHUNKS + NBUF - 1) // NBUF


def _gather_body(table_hbm, rows_hbm, cols_hbm, out_hbm, idx_v, bufs, gsem,
                 wsem):
    wid = lax.axis_index("s") * NC + lax.axis_index("c")
    base = wid * R_PER_W
    pltpu.sync_copy(rows_hbm.at[0, pl.ds(base, R_PER_W)],
                    idx_v.at[pl.ds(0, R_PER_W)])
    pltpu.sync_copy(cols_hbm.at[0, pl.ds(base, R_PER_W)],
                    idx_v.at[pl.ds(R_PER_W, R_PER_W)])

    def start_gathers(g, b):
        pltpu.async_copy(
            table_hbm.at[idx_v.at[pl.ds(g * CH2, CH2)]],
            bufs.at[b, pl.ds(0, CH2), pl.ds(0, D)], gsem.at[b])
        pltpu.async_copy(
            table_hbm.at[idx_v.at[pl.ds(R_PER_W + g * CH2, CH2)]],
            bufs.at[b, pl.ds(0, CH2), pl.ds(D, D)], gsem.at[b])

    def wait_gathers(b):
        pltpu.make_async_copy(
            out_hbm.at[0, pl.ds(0, CH2)], bufs.at[b], gsem.at[b]).wait()

    def start_write(g, b):
        pltpu.make_async_copy(
            bufs.at[b], out_hbm.at[0, pl.ds(base + g * CH2, CH2)],
            wsem.at[b]).start()

    def wait_write(b):
        pltpu.make_async_copy(
            bufs.at[b], out_hbm.at[0, pl.ds(0, CH2)], wsem.at[b]).wait()

    start_gathers(0, 0)
    start_gathers(1, 1)

    def round_fn(r, carry):
        for b in range(NBUF):
            g = r * NBUF + b

            @pl.when(g < N_CHUNKS)
            def _():
                wait_gathers(b)
                start_write(g, b)
                h = g + 2
                bh = (b + 2) % NBUF

                @pl.when(h < N_CHUNKS)
                def _():
                    @pl.when(h >= NBUF)
                    def _():
                        wait_write(bh)

                    start_gathers(h, bh)
        return carry

    lax.fori_loop(0, N_ROUNDS, round_fn, 0)
    for b in range(NBUF):
        wait_write(b)


@jax.jit
def kernel(rows, cols, cos_sin):
    mesh = plsc.VectorSubcoreMesh(core_axis_name="c", subcore_axis_name="s")
    return pl.kernel(
        _gather_body,
        mesh=mesh,
        out_type=jax.ShapeDtypeStruct((1, R, 2 * D), jnp.float32),
        scratch_types=[
            pltpu.VMEM((2 * R_PER_W,), jnp.int32),
            pltpu.VMEM((NBUF, CH2, 2 * D), jnp.float32),
            pltpu.SemaphoreType.DMA((NBUF,)),
            pltpu.SemaphoreType.DMA((NBUF,)),
        ],
    )(cos_sin, rows, cols)


# speedup vs baseline: 1.0030x; 1.0030x over previous
import jax
import jax.numpy as jnp
from jax import lax
from jax.experimental import pallas as pl
from jax.experimental.pallas import tpu as pltpu
from jax.experimental.pallas import tpu_sc as plsc

D = 2048
R = 8192
NC = 2
NS = 16
NW = NC * NS
R_PER_W = R // NW
CH2 = 8
NBUF = 3
N_CHUNKS = R_PER_W // CH2
N_ROUNDS = (N_CHUNKS + NBUF - 1) // NBUF


def _gather_body(table_hbm, rows_hbm, cols_hbm, out_hbm, idx_v, bufs, gsem,
                 wsem):
    wid = lax.axis_index("s") * NC + lax.axis_index("c")
    base = wid * R_PER_W
    pltpu.sync_copy(rows_hbm.at[0, pl.ds(base, R_PER_W)],
                    idx_v.at[pl.ds(0, R_PER_W)])
    pltpu.sync_copy(cols_hbm.at[0, pl.ds(base, R_PER_W)],
                    idx_v.at[pl.ds(R_PER_W, R_PER_W)])

    def start_gathers(g, b):
        pltpu.async_copy(
            table_hbm.at[idx_v.at[pl.ds(g * CH2, CH2)]],
            bufs.at[b, pl.ds(0, CH2), pl.ds(0, D)], gsem.at[b])
        pltpu.async_copy(
            table_hbm.at[idx_v.at[pl.ds(R_PER_W + g * CH2, CH2)]],
            bufs.at[b, pl.ds(0, CH2), pl.ds(D, D)], gsem.at[b])

    def wait_gathers(b):
        pltpu.make_async_copy(
            out_hbm.at[0, pl.ds(0, CH2)], bufs.at[b], gsem.at[b]).wait()

    for b in range(NBUF):
        start_gathers(b, b)

    def round_fn(r, carry):
        for b in range(NBUF):
            g = r * NBUF + b

            @pl.when(g < N_CHUNKS)
            def _():
                wait_gathers(b)
                cp = pltpu.make_async_copy(
                    bufs.at[b], out_hbm.at[0, pl.ds(base + g * CH2, CH2)],
                    wsem.at[b])
                cp.start()
                cp.wait()

                @pl.when(g + NBUF < N_CHUNKS)
                def _():
                    start_gathers(g + NBUF, b)
        return carry

    lax.fori_loop(0, N_ROUNDS, round_fn, 0)


@jax.jit
def kernel(rows, cols, cos_sin):
    mesh = plsc.VectorSubcoreMesh(core_axis_name="c", subcore_axis_name="s")
    return pl.kernel(
        _gather_body,
        mesh=mesh,
        out_type=jax.ShapeDtypeStruct((1, R, 2 * D), jnp.float32),
        scratch_types=[
            pltpu.VMEM((2 * R_PER_W,), jnp.int32),
            pltpu.VMEM((NBUF, CH2, 2 * D), jnp.float32),
            pltpu.SemaphoreType.DMA((NBUF,)),
            pltpu.SemaphoreType.DMA((NBUF,)),
        ],
    )(cos_sin, rows, cols)
